# baseline (speedup 1.0000x reference)
import functools

import jax
import jax.numpy as jnp
from jax import lax
from jax.experimental import pallas as pl
from jax.experimental.pallas import tpu as pltpu
from jax.experimental.pallas import tpu_sc as plsc

VOCAB = 1000000
EMBED = 32
B, F, P = 4096, 26, 20
BF = B * F
NW = 32
N = 64
NB = B // N
NCHUNK = F * NB
CPW = NCHUNK // NW
IDS = N * P
HALF = EMBED // 2
PADC = 128
RSUB = PADC // EMBED


def _body(ids_hbm, table_hbm, out_hbm,
          idx0, idx1, sidx0, sidx1, acc0, acc1, sem0, sem1):
    wid = lax.axis_index("s") * 2 + lax.axis_index("c")
    q0 = wid * CPW
    zero16 = jnp.zeros((16,), jnp.float32)

    def issue(q, idx_v, sidx_v, acc_v, sem):
        f = q // NB
        b0 = (q % NB) * N
        pltpu.sync_copy(ids_hbm.at[pl.ds(f * P, P), pl.ds(b0, N)], idx_v)

        def zero_body(n, carry):
            acc_v[n, pl.ds(0, 16)] = zero16
            acc_v[n, pl.ds(16, 16)] = zero16
            return carry

        lax.fori_loop(0, N, zero_body, 0)
        for p in range(P):
            for j in range(N // 16):
                g = idx_v[p, pl.ds(j * 16, 16)]
                sidx_v[pl.ds(p * N + j * 16, 16)] = g * RSUB
        for p in range(P):
            pltpu.async_copy(
                table_hbm.at[sidx_v.at[pl.ds(p * N, N)]],
                acc_v, sem, add=True)

    def drain(sidx_v, acc_v, sem):
        for p in range(P):
            pltpu.make_async_copy(
                table_hbm.at[sidx_v.at[pl.ds(p * N, N)]],
                acc_v, sem).wait()

    def compute(q, idx_v, acc_v):
        def blk_body(jj, carry2):
            n0 = jj * 16
            cnt = jnp.zeros((16,), jnp.float32)
            for p in range(P):
                g = idx_v[p, pl.ds(n0, 16)]
                cnt = cnt + jnp.where(g != 0, 1.0, 0.0)
            inv16 = 1.0 / jnp.maximum(cnt, 1.0)
            for l in range(16):
                inv_s = inv16[l]
                for h in range(2):
                    acc_v[n0 + l, pl.ds(h * HALF, HALF)] = (
                        acc_v[n0 + l, pl.ds(h * HALF, HALF)] * inv_s)
            return carry2

        lax.fori_loop(0, N // 16, blk_body, 0)
        f = q // NB
        b0 = (q % NB) * N
        pltpu.sync_copy(acc_v, out_hbm.at[pl.ds(b0, N), f])

    issue(q0, idx0, sidx0, acc0, sem0)

    def pair_body(i, carry):
        c0 = q0 + i * 2
        issue(c0 + 1, idx1, sidx1, acc1, sem1)
        drain(sidx0, acc0, sem0)
        compute(c0, idx0, acc0)

        @pl.when(i * 2 + 2 < CPW)
        def _():
            issue(c0 + 2, idx0, sidx0, acc0, sem0)

        drain(sidx1, acc1, sem1)
        compute(c0 + 1, idx1, acc1)
        return carry

    lax.fori_loop(0, CPW // 2, pair_body, 0)


BK = 512


def _tpose_body(x_ref, o_ref):
    x = x_ref[...].T
    o_ref[...] = jnp.concatenate(
        [x, jnp.zeros((BK, PADC - EMBED), jnp.float32)], axis=1)


def _stage_table(table):
    t1 = table.T
    grid = (VOCAB + BK - 1) // BK
    tp = pl.pallas_call(
        _tpose_body,
        grid=(grid,),
        in_specs=[pl.BlockSpec((EMBED, BK), lambda j: (0, j))],
        out_specs=pl.BlockSpec((BK, PADC), lambda j: (j, 0)),
        out_shape=jax.ShapeDtypeStruct((VOCAB, PADC), jnp.float32),
    )(t1)
    return tp.reshape(VOCAB * RSUB, EMBED)


def kernel(path_ids, table):
    ids_fp_b = jnp.transpose(path_ids, (1, 2, 0)).reshape(F * P, B)
    tab4 = _stage_table(table)
    mesh = plsc.VectorSubcoreMesh(core_axis_name="c", subcore_axis_name="s")
    run = functools.partial(
        pl.kernel,
        out_type=jax.ShapeDtypeStruct((B, F, EMBED), jnp.float32),
        mesh=mesh,
        compiler_params=pltpu.CompilerParams(use_tc_tiling_on_sc=False),
        scratch_types=[
            pltpu.VMEM((P, N), jnp.int32),
            pltpu.VMEM((P, N), jnp.int32),
            pltpu.VMEM((IDS,), jnp.int32),
            pltpu.VMEM((IDS,), jnp.int32),
            pltpu.VMEM((N, EMBED), jnp.float32),
            pltpu.VMEM((N, EMBED), jnp.float32),
            pltpu.SemaphoreType.DMA,
            pltpu.SemaphoreType.DMA,
        ],
    )(_body)
    return run(ids_fp_b, tab4)


# speedup vs baseline: 2.7093x; 2.7093x over previous
import functools

import jax
import jax.numpy as jnp
from jax import lax
from jax.experimental import pallas as pl
from jax.experimental.pallas import tpu as pltpu
from jax.experimental.pallas import tpu_sc as plsc

VOCAB = 1000000
EMBED = 32
B, F, P = 4096, 26, 20
BF = B * F
NW = 32
N = 64
NB = B // N
NCHUNK = F * NB
CPW = NCHUNK // NW
IDS = N * P
HALF = EMBED // 2
PADC = 128
RSUB = PADC // EMBED


def _body(ids_hbm, table_hbm, out_hbm,
          idx0, idx1, sidx0, sidx1, acc0, acc1, sem0, sem1):
    wid = lax.axis_index("s") * 2 + lax.axis_index("c")
    q0 = wid * CPW
    zero16 = jnp.zeros((16,), jnp.float32)

    def issue(q, idx_v, sidx_v, acc_v, sem):
        f = q // NB
        b0 = (q % NB) * N
        pltpu.sync_copy(ids_hbm.at[pl.ds(f * P, P), pl.ds(b0, N)], idx_v)

        def zero_body(n, carry):
            acc_v[n, pl.ds(0, 16)] = zero16
            acc_v[n, pl.ds(16, 16)] = zero16
            return carry

        lax.fori_loop(0, N, zero_body, 0)
        for p in range(P):
            for j in range(N // 16):
                g = idx_v[p, pl.ds(j * 16, 16)]
                sidx_v[pl.ds(p * N + j * 16, 16)] = g * RSUB
        for p in range(P):
            pltpu.async_copy(
                table_hbm.at[sidx_v.at[pl.ds(p * N, N)]],
                acc_v, sem, add=True)

    def drain(sidx_v, acc_v, sem):
        for p in range(P):
            pltpu.make_async_copy(
                table_hbm.at[sidx_v.at[pl.ds(p * N, N)]],
                acc_v, sem).wait()

    def compute(q, idx_v, acc_v):
        def blk_body(jj, carry2):
            n0 = jj * 16
            cnt = jnp.zeros((16,), jnp.float32)
            for p in range(P):
                g = idx_v[p, pl.ds(n0, 16)]
                cnt = cnt + jnp.where(g != 0, 1.0, 0.0)
            inv16 = 1.0 / jnp.maximum(cnt, 1.0)
            for l in range(16):
                inv_s = inv16[l]
                for h in range(2):
                    acc_v[n0 + l, pl.ds(h * HALF, HALF)] = (
                        acc_v[n0 + l, pl.ds(h * HALF, HALF)] * inv_s)
            return carry2

        lax.fori_loop(0, N // 16, blk_body, 0)
        f = q // NB
        b0 = (q % NB) * N
        pltpu.sync_copy(acc_v, out_hbm.at[pl.ds(b0, N), f])

    issue(q0, idx0, sidx0, acc0, sem0)

    def pair_body(i, carry):
        c0 = q0 + i * 2
        issue(c0 + 1, idx1, sidx1, acc1, sem1)
        drain(sidx0, acc0, sem0)
        compute(c0, idx0, acc0)

        @pl.when(i * 2 + 2 < CPW)
        def _():
            issue(c0 + 2, idx0, sidx0, acc0, sem0)

        drain(sidx1, acc1, sem1)
        compute(c0 + 1, idx1, acc1)
        return carry

    lax.fori_loop(0, CPW // 2, pair_body, 0)


BK = 4096


def _tpose_body(x_ref, o_ref):
    x = x_ref[...].T
    o_ref[...] = jnp.concatenate(
        [x, jnp.zeros((BK, PADC - EMBED), jnp.float32)], axis=1)


def _stage_table(table):
    t1 = table.T
    grid = (VOCAB + BK - 1) // BK
    tp = pl.pallas_call(
        _tpose_body,
        grid=(grid,),
        in_specs=[pl.BlockSpec((EMBED, BK), lambda j: (0, j))],
        out_specs=pl.BlockSpec((BK, PADC), lambda j: (j, 0)),
        out_shape=jax.ShapeDtypeStruct((VOCAB, PADC), jnp.float32),
    )(t1)
    return tp.reshape(VOCAB * RSUB, EMBED)


def kernel(path_ids, table):
    ids_fp_b = jnp.transpose(path_ids, (1, 2, 0)).reshape(F * P, B)
    tab4 = _stage_table(table)
    mesh = plsc.VectorSubcoreMesh(core_axis_name="c", subcore_axis_name="s")
    run = functools.partial(
        pl.kernel,
        out_type=jax.ShapeDtypeStruct((B, F, EMBED), jnp.float32),
        mesh=mesh,
        compiler_params=pltpu.CompilerParams(use_tc_tiling_on_sc=False),
        scratch_types=[
            pltpu.VMEM((P, N), jnp.int32),
            pltpu.VMEM((P, N), jnp.int32),
            pltpu.VMEM((IDS,), jnp.int32),
            pltpu.VMEM((IDS,), jnp.int32),
            pltpu.VMEM((N, EMBED), jnp.float32),
            pltpu.VMEM((N, EMBED), jnp.float32),
            pltpu.SemaphoreType.DMA,
            pltpu.SemaphoreType.DMA,
        ],
    )(_body)
    return run(ids_fp_b, tab4)


# speedup vs baseline: 3.1404x; 1.1591x over previous
import functools

import jax
import jax.numpy as jnp
from jax import lax
from jax.experimental import pallas as pl
from jax.experimental.pallas import tpu as pltpu
from jax.experimental.pallas import tpu_sc as plsc

VOCAB = 1000000
EMBED = 32
B, F, P = 4096, 26, 20
BF = B * F
NW = 32
N = 64
NB = B // N
NCHUNK = F * NB
CPW = NCHUNK // NW
IDS = N * P
HALF = EMBED // 2
PADC = 128
RSUB = PADC // EMBED


def _body(ids_hbm, table_hbm, out_hbm,
          idx0, idx1, sidx0, sidx1, acc0, acc1, sem0, sem1):
    wid = lax.axis_index("s") * 2 + lax.axis_index("c")
    q0 = wid * CPW
    zero16 = jnp.zeros((16,), jnp.float32)

    def issue(q, idx_v, sidx_v, acc_v, sem):
        f = q // NB
        b0 = (q % NB) * N
        pltpu.sync_copy(ids_hbm.at[pl.ds(f * P, P), pl.ds(b0, N)], idx_v)

        def zero_body(n, carry):
            acc_v[n, pl.ds(0, 16)] = zero16
            acc_v[n, pl.ds(16, 16)] = zero16
            return carry

        lax.fori_loop(0, N, zero_body, 0)
        for p in range(P):
            for j in range(N // 16):
                g = idx_v[p, pl.ds(j * 16, 16)]
                sidx_v[pl.ds(p * N + j * 16, 16)] = g * RSUB
        for p in range(P):
            pltpu.async_copy(
                table_hbm.at[sidx_v.at[pl.ds(p * N, N)]],
                acc_v, sem, add=True)

    def drain(sidx_v, acc_v, sem):
        for p in range(P):
            pltpu.make_async_copy(
                table_hbm.at[sidx_v.at[pl.ds(p * N, N)]],
                acc_v, sem).wait()

    def compute(q, idx_v, acc_v):
        def blk_body(jj, carry2):
            n0 = jj * 16
            cnt = jnp.zeros((16,), jnp.float32)
            for p in range(P):
                g = idx_v[p, pl.ds(n0, 16)]
                cnt = cnt + jnp.where(g != 0, 1.0, 0.0)
            inv16 = 1.0 / jnp.maximum(cnt, 1.0)
            for l in range(16):
                inv_s = inv16[l]
                for h in range(2):
                    acc_v[n0 + l, pl.ds(h * HALF, HALF)] = (
                        acc_v[n0 + l, pl.ds(h * HALF, HALF)] * inv_s)
            return carry2

        lax.fori_loop(0, N // 16, blk_body, 0)
        f = q // NB
        b0 = (q % NB) * N
        pltpu.sync_copy(acc_v, out_hbm.at[pl.ds(b0, N), f])

    issue(q0, idx0, sidx0, acc0, sem0)

    def pair_body(i, carry):
        c0 = q0 + i * 2
        issue(c0 + 1, idx1, sidx1, acc1, sem1)
        drain(sidx0, acc0, sem0)
        compute(c0, idx0, acc0)

        @pl.when(i * 2 + 2 < CPW)
        def _():
            issue(c0 + 2, idx0, sidx0, acc0, sem0)

        drain(sidx1, acc1, sem1)
        compute(c0 + 1, idx1, acc1)
        return carry

    lax.fori_loop(0, CPW // 2, pair_body, 0)


BK = 8192


def _tpose_body(x_ref, o_ref):
    x = x_ref[...].T
    o_ref[...] = jnp.concatenate(
        [x, jnp.zeros((BK, PADC - EMBED), jnp.float32)], axis=1)


def _stage_table(table):
    t1 = table.T
    grid = (VOCAB + BK - 1) // BK
    tp = pl.pallas_call(
        _tpose_body,
        grid=(grid,),
        in_specs=[pl.BlockSpec((EMBED, BK), lambda j: (0, j))],
        out_specs=pl.BlockSpec((BK, PADC), lambda j: (j, 0)),
        out_shape=jax.ShapeDtypeStruct((VOCAB, PADC), jnp.float32),
    )(t1)
    return tp.reshape(VOCAB * RSUB, EMBED)


def kernel(path_ids, table):
    ids_fp_b = jnp.transpose(path_ids, (1, 2, 0)).reshape(F * P, B)
    tab4 = _stage_table(table)
    mesh = plsc.VectorSubcoreMesh(core_axis_name="c", subcore_axis_name="s")
    run = functools.partial(
        pl.kernel,
        out_type=jax.ShapeDtypeStruct((B, F, EMBED), jnp.float32),
        mesh=mesh,
        compiler_params=pltpu.CompilerParams(use_tc_tiling_on_sc=False),
        scratch_types=[
            pltpu.VMEM((P, N), jnp.int32),
            pltpu.VMEM((P, N), jnp.int32),
            pltpu.VMEM((IDS,), jnp.int32),
            pltpu.VMEM((IDS,), jnp.int32),
            pltpu.VMEM((N, EMBED), jnp.float32),
            pltpu.VMEM((N, EMBED), jnp.float32),
            pltpu.SemaphoreType.DMA,
            pltpu.SemaphoreType.DMA,
        ],
    )(_body)
    return run(ids_fp_b, tab4)


# speedup vs baseline: 3.3857x; 1.0781x over previous
import functools

import jax
import jax.numpy as jnp
from jax import lax
from jax.experimental import pallas as pl
from jax.experimental.pallas import tpu as pltpu
from jax.experimental.pallas import tpu_sc as plsc

VOCAB = 1000000
EMBED = 32
B, F, P = 4096, 26, 20
BF = B * F
NW = 32
N = 64
NB = B // N
NCHUNK = F * NB
CPW = NCHUNK // NW
IDS = N * P
HALF = EMBED // 2
PADC = 128
RSUB = PADC // EMBED


def _body(ids_hbm, table_hbm, out_hbm,
          idx0, idx1, sidx0, sidx1, acc0, acc1, sem0, sem1):
    wid = lax.axis_index("s") * 2 + lax.axis_index("c")
    q0 = wid * CPW
    zero16 = jnp.zeros((16,), jnp.float32)

    def issue(q, idx_v, sidx_v, acc_v, sem):
        f = q // NB
        b0 = (q % NB) * N
        pltpu.sync_copy(ids_hbm.at[pl.ds(f * P, P), pl.ds(b0, N)], idx_v)

        def zero_body(n, carry):
            acc_v[n, pl.ds(0, 16)] = zero16
            acc_v[n, pl.ds(16, 16)] = zero16
            return carry

        lax.fori_loop(0, N, zero_body, 0)
        for p in range(P):
            for j in range(N // 16):
                g = idx_v[p, pl.ds(j * 16, 16)]
                sidx_v[pl.ds(p * N + j * 16, 16)] = g * RSUB
        for p in range(P):
            pltpu.async_copy(
                table_hbm.at[sidx_v.at[pl.ds(p * N, N)]],
                acc_v, sem, add=True)

    def drain(sidx_v, acc_v, sem):
        for p in range(P):
            pltpu.make_async_copy(
                table_hbm.at[sidx_v.at[pl.ds(p * N, N)]],
                acc_v, sem).wait()

    def compute(q, idx_v, acc_v):
        def blk_body(jj, carry2):
            n0 = jj * 16
            cnt = jnp.zeros((16,), jnp.float32)
            for p in range(P):
                g = idx_v[p, pl.ds(n0, 16)]
                cnt = cnt + jnp.where(g != 0, 1.0, 0.0)
            inv16 = 1.0 / jnp.maximum(cnt, 1.0)
            for l in range(16):
                inv_s = inv16[l]
                for h in range(2):
                    acc_v[n0 + l, pl.ds(h * HALF, HALF)] = (
                        acc_v[n0 + l, pl.ds(h * HALF, HALF)] * inv_s)
            return carry2

        lax.fori_loop(0, N // 16, blk_body, 0)
        f = q // NB
        b0 = (q % NB) * N
        pltpu.sync_copy(acc_v, out_hbm.at[pl.ds(b0, N), f])

    issue(q0, idx0, sidx0, acc0, sem0)

    def pair_body(i, carry):
        c0 = q0 + i * 2
        issue(c0 + 1, idx1, sidx1, acc1, sem1)
        drain(sidx0, acc0, sem0)
        compute(c0, idx0, acc0)

        @pl.when(i * 2 + 2 < CPW)
        def _():
            issue(c0 + 2, idx0, sidx0, acc0, sem0)

        drain(sidx1, acc1, sem1)
        compute(c0 + 1, idx1, acc1)
        return carry

    lax.fori_loop(0, CPW // 2, pair_body, 0)


BK = 16384


def _tpose_body(x_ref, o_ref):
    x = x_ref[...].T
    o_ref[...] = jnp.concatenate(
        [x, jnp.zeros((BK, PADC - EMBED), jnp.float32)], axis=1)


def _stage_table(table):
    t1 = table.T
    grid = (VOCAB + BK - 1) // BK
    tp = pl.pallas_call(
        _tpose_body,
        grid=(grid,),
        in_specs=[pl.BlockSpec((EMBED, BK), lambda j: (0, j))],
        out_specs=pl.BlockSpec((BK, PADC), lambda j: (j, 0)),
        out_shape=jax.ShapeDtypeStruct((VOCAB, PADC), jnp.float32),
    )(t1)
    return tp.reshape(VOCAB * RSUB, EMBED)


def kernel(path_ids, table):
    ids_fp_b = jnp.transpose(path_ids, (1, 2, 0)).reshape(F * P, B)
    tab4 = _stage_table(table)
    mesh = plsc.VectorSubcoreMesh(core_axis_name="c", subcore_axis_name="s")
    run = functools.partial(
        pl.kernel,
        out_type=jax.ShapeDtypeStruct((B, F, EMBED), jnp.float32),
        mesh=mesh,
        compiler_params=pltpu.CompilerParams(use_tc_tiling_on_sc=False),
        scratch_types=[
            pltpu.VMEM((P, N), jnp.int32),
            pltpu.VMEM((P, N), jnp.int32),
            pltpu.VMEM((IDS,), jnp.int32),
            pltpu.VMEM((IDS,), jnp.int32),
            pltpu.VMEM((N, EMBED), jnp.float32),
            pltpu.VMEM((N, EMBED), jnp.float32),
            pltpu.SemaphoreType.DMA,
            pltpu.SemaphoreType.DMA,
        ],
    )(_body)
    return run(ids_fp_b, tab4)


# speedup vs baseline: 3.4247x; 1.0115x over previous
import functools

import jax
import jax.numpy as jnp
from jax import lax
from jax.experimental import pallas as pl
from jax.experimental.pallas import tpu as pltpu
from jax.experimental.pallas import tpu_sc as plsc

VOCAB = 1000000
EMBED = 32
B, F, P = 4096, 26, 20
BF = B * F
NW = 32
N = 64
NB = B // N
NCHUNK = F * NB
CPW = NCHUNK // NW
IDS = N * P
HALF = EMBED // 2
PADC = 128
RSUB = PADC // EMBED


def _body(ids_hbm, table_hbm, out_hbm,
          idx0, idx1, sidx0, sidx1, acc0, acc1, sem0, sem1):
    wid = lax.axis_index("s") * 2 + lax.axis_index("c")
    q0 = wid * CPW
    zero16 = jnp.zeros((16,), jnp.float32)

    def issue(q, idx_v, sidx_v, acc_v, sem):
        f = q // NB
        b0 = (q % NB) * N
        pltpu.sync_copy(ids_hbm.at[pl.ds(f * P, P), pl.ds(b0, N)], idx_v)

        def zero_body(n, carry):
            acc_v[n, pl.ds(0, 16)] = zero16
            acc_v[n, pl.ds(16, 16)] = zero16
            return carry

        lax.fori_loop(0, N, zero_body, 0)
        for p in range(P):
            for j in range(N // 16):
                g = idx_v[p, pl.ds(j * 16, 16)]
                sidx_v[pl.ds(p * N + j * 16, 16)] = g * RSUB
        for p in range(P):
            pltpu.async_copy(
                table_hbm.at[sidx_v.at[pl.ds(p * N, N)]],
                acc_v, sem, add=True)

    def drain(sidx_v, acc_v, sem):
        for p in range(P):
            pltpu.make_async_copy(
                table_hbm.at[sidx_v.at[pl.ds(p * N, N)]],
                acc_v, sem).wait()

    def compute(q, idx_v, acc_v):
        def blk_body(jj, carry2):
            n0 = jj * 16
            cnt = jnp.zeros((16,), jnp.float32)
            for p in range(P):
                g = idx_v[p, pl.ds(n0, 16)]
                cnt = cnt + jnp.where(g != 0, 1.0, 0.0)
            inv16 = 1.0 / jnp.maximum(cnt, 1.0)
            for l in range(16):
                inv_s = inv16[l]
                for h in range(2):
                    acc_v[n0 + l, pl.ds(h * HALF, HALF)] = (
                        acc_v[n0 + l, pl.ds(h * HALF, HALF)] * inv_s)
            return carry2

        lax.fori_loop(0, N // 16, blk_body, 0)
        f = q // NB
        b0 = (q % NB) * N
        pltpu.sync_copy(acc_v, out_hbm.at[pl.ds(b0, N), f])

    issue(q0, idx0, sidx0, acc0, sem0)

    def pair_body(i, carry):
        c0 = q0 + i * 2
        issue(c0 + 1, idx1, sidx1, acc1, sem1)
        drain(sidx0, acc0, sem0)
        compute(c0, idx0, acc0)

        @pl.when(i * 2 + 2 < CPW)
        def _():
            issue(c0 + 2, idx0, sidx0, acc0, sem0)

        drain(sidx1, acc1, sem1)
        compute(c0 + 1, idx1, acc1)
        return carry

    lax.fori_loop(0, CPW // 2, pair_body, 0)


BK = 32768


def _tpose_body(x_ref, o_ref):
    x = x_ref[...].T
    o_ref[...] = jnp.concatenate(
        [x, jnp.zeros((BK, PADC - EMBED), jnp.float32)], axis=1)


def _stage_table(table):
    t1 = table.T
    grid = (VOCAB + BK - 1) // BK
    tp = pl.pallas_call(
        _tpose_body,
        grid=(grid,),
        in_specs=[pl.BlockSpec((EMBED, BK), lambda j: (0, j))],
        out_specs=pl.BlockSpec((BK, PADC), lambda j: (j, 0)),
        out_shape=jax.ShapeDtypeStruct((VOCAB, PADC), jnp.float32),
    )(t1)
    return tp.reshape(VOCAB * RSUB, EMBED)


def kernel(path_ids, table):
    ids_fp_b = jnp.transpose(path_ids, (1, 2, 0)).reshape(F * P, B)
    tab4 = _stage_table(table)
    mesh = plsc.VectorSubcoreMesh(core_axis_name="c", subcore_axis_name="s")
    run = functools.partial(
        pl.kernel,
        out_type=jax.ShapeDtypeStruct((B, F, EMBED), jnp.float32),
        mesh=mesh,
        compiler_params=pltpu.CompilerParams(use_tc_tiling_on_sc=False),
        scratch_types=[
            pltpu.VMEM((P, N), jnp.int32),
            pltpu.VMEM((P, N), jnp.int32),
            pltpu.VMEM((IDS,), jnp.int32),
            pltpu.VMEM((IDS,), jnp.int32),
            pltpu.VMEM((N, EMBED), jnp.float32),
            pltpu.VMEM((N, EMBED), jnp.float32),
            pltpu.SemaphoreType.DMA,
            pltpu.SemaphoreType.DMA,
        ],
    )(_body)
    return run(ids_fp_b, tab4)
